# Initial kernel scaffold; baseline (speedup 1.0000x reference)
import jax
import jax.numpy as jnp
from jax.experimental import pallas as pl


def kernel(centroid_embs, finetune_embs, assign_val, ft_assign_val, edge_weight, assign_idx, ft_assign_idx, edge_index):
    raise NotImplementedError("write your pallas kernel here")


# speedup vs baseline: 1.7934x; 1.7934x over previous
import functools

import jax
import jax.numpy as jnp
from jax import lax
from jax.experimental import pallas as pl
from jax.experimental.pallas import tpu as pltpu
from jax.experimental.pallas import tpu_sc as plsc

NC = 2
NS = 16
NW = NC * NS
L = 16

N = 50000
D = 128
C1 = 8192
C2 = 1024
K1 = 8
K2 = 4
E = 800000

SLABS = 4
SW = D // SLABS
N_PAD = 50176
E_PAD = 800768

_mesh = plsc.VectorSubcoreMesh(core_axis_name="c", subcore_axis_name="s")


def _wid():
    return lax.axis_index("s") * NC + lax.axis_index("c")


def _splat_f32(ref, i):
    v = ref[pl.ds((i // L) * L, L)]
    idx = jnp.full((L,), i % L, dtype=jnp.int32)
    dnums = lax.GatherDimensionNumbers(
        offset_dims=(), collapsed_slice_dims=(0,), start_index_map=(0,))
    return lax.gather(v, idx[:, None], dnums, (1,),
                      mode=lax.GatherScatterMode.PROMISE_IN_BOUNDS)


RF_RC = 16
RF_STEPS = C1 // NW // RF_RC


@functools.partial(
    pl.kernel,
    out_type=jax.ShapeDtypeStruct((C1, D), jnp.float32),
    mesh=_mesh,
    scratch_types=[
        pltpu.VMEM((RF_RC, D), jnp.float32),
        pltpu.VMEM((RF_RC * K2,), jnp.int32),
        pltpu.VMEM((RF_RC * K2,), jnp.float32),
        pltpu.VMEM((RF_RC * K2, D), jnp.float32),
        pltpu.VMEM((RF_RC, D), jnp.float32),
        pltpu.SemaphoreType.DMA,
    ],
)
def _refine_kernel(cen_hbm, ft_hbm, val_hbm, idx_hbm, out_hbm,
                   cen_v, idx_v, val_v, rows_v, out_v, sem):
    base = _wid() * (RF_RC * RF_STEPS)

    @pl.loop(0, RF_STEPS)
    def _(t):
        off = base + t * RF_RC
        pltpu.sync_copy(cen_hbm.at[pl.ds(off, RF_RC), :], cen_v)
        pltpu.sync_copy(idx_hbm.at[pl.ds(off * K2, RF_RC * K2)], idx_v)
        pltpu.sync_copy(val_hbm.at[pl.ds(off * K2, RF_RC * K2)], val_v)
        pltpu.async_copy(ft_hbm.at[idx_v], rows_v, sem).wait()
        for r in range(RF_RC):
            w = [_splat_f32(val_v, r * K2 + k) for k in range(K2)]
            for d in range(D // L):
                sl = pl.ds(d * L, L)
                acc = cen_v[r, sl]
                for k in range(K2):
                    acc = acc + w[k] * rows_v[r * K2 + k, sl]
                out_v[r, sl] = acc
        pltpu.sync_copy(out_v, out_hbm.at[pl.ds(off, RF_RC), :])


X0_EC = 16
X0_STEPS = N_PAD // NW // X0_EC


@functools.partial(
    pl.kernel,
    out_type=jax.ShapeDtypeStruct((SLABS, N_PAD, SW), jnp.float32),
    mesh=_mesh,
    scratch_types=[
        pltpu.VMEM((X0_EC * K1,), jnp.int32),
        pltpu.VMEM((X0_EC * K1,), jnp.float32),
        pltpu.VMEM((X0_EC * K1, D), jnp.float32),
        pltpu.VMEM((SLABS, X0_EC, SW), jnp.float32),
        pltpu.SemaphoreType.DMA,
    ],
    compiler_params=pltpu.CompilerParams(use_tc_tiling_on_sc=False),
)
def _compose_kernel(ref_hbm, val_hbm, idx_hbm, out_hbm,
                    idx_v, val_v, rows_v, out_v, sem):
    base = _wid() * (X0_EC * X0_STEPS)

    @pl.loop(0, X0_STEPS)
    def _(t):
        off = base + t * X0_EC
        pltpu.sync_copy(idx_hbm.at[pl.ds(off * K1, X0_EC * K1)], idx_v)
        pltpu.sync_copy(val_hbm.at[pl.ds(off * K1, X0_EC * K1)], val_v)
        pltpu.async_copy(ref_hbm.at[idx_v], rows_v, sem).wait()
        for e in range(X0_EC):
            w = [_splat_f32(val_v, e * K1 + k) for k in range(K1)]
            for d in range(D // L):
                sl = pl.ds((d % 2) * L, L)
                acc = w[0] * rows_v[e * K1, pl.ds(d * L, L)]
                for k in range(1, K1):
                    acc = acc + w[k] * rows_v[e * K1 + k, pl.ds(d * L, L)]
                out_v[d // 2, e, sl] = acc
        for s in range(SLABS):
            pltpu.sync_copy(out_v.at[s],
                            out_hbm.at[s, pl.ds(off, X0_EC), :])


PR_EC = 128
PR_STEPS = E_PAD // NS // PR_EC
ZB_ROWS = 392


@functools.partial(
    pl.kernel,
    out_type=jax.ShapeDtypeStruct((SLABS, N_PAD, SW), jnp.float32),
    mesh=_mesh,
    scratch_types=[
        pltpu.VMEM((PR_EC,), jnp.int32),
        pltpu.VMEM((1, PR_EC), jnp.int32),
        pltpu.VMEM((PR_EC,), jnp.float32),
        pltpu.VMEM((PR_EC,), jnp.int32),
        pltpu.VMEM((PR_EC, SW), jnp.float32),
        pltpu.VMEM((ZB_ROWS, SW), jnp.float32),
        pltpu.VMEM_SHARED((N_PAD, SW), jnp.float32),
        pltpu.SemaphoreType.DMA,
    ],
    compiler_params=pltpu.CompilerParams(use_tc_tiling_on_sc=False),
)
def _prop_kernel(x_hbm, src_hbm, dst_hbm, w_hbm, out_hbm,
                 src_v, dst_v, w_v, gidx_v, rows_v, zb_v, acc_sh, sem):
    c = lax.axis_index("c")
    sid = lax.axis_index("s")
    ebase = sid * (PR_EC * PR_STEPS)
    rows_per_tile = N_PAD // NS
    r0 = sid * rows_per_tile

    for i in range(ZB_ROWS):
        for j in range(SW // L):
            zb_v[i, pl.ds(j * L, L)] = jnp.zeros((L,), jnp.float32)

    for j in range(2):
        s = c * 2 + j
        for i in range(rows_per_tile // ZB_ROWS):
            pltpu.sync_copy(zb_v,
                            acc_sh.at[pl.ds(r0 + i * ZB_ROWS, ZB_ROWS), :])
        plsc.subcore_barrier()

        sbase = s * N_PAD

        @pl.loop(0, PR_STEPS)
        def _(t):
            off = ebase + t * PR_EC
            pltpu.sync_copy(src_hbm.at[pl.ds(off, PR_EC)], src_v)
            pltpu.sync_copy(dst_hbm.at[pl.ds(off, PR_EC)], dst_v.at[0])
            pltpu.sync_copy(w_hbm.at[pl.ds(off, PR_EC)], w_v)
            sb = jnp.full((L,), sbase, dtype=jnp.int32)
            for i in range(PR_EC // L):
                sl = pl.ds(i * L, L)
                gidx_v[sl] = src_v[sl] + sb
            pltpu.async_copy(x_hbm.at[gidx_v], rows_v, sem).wait()
            for e in range(PR_EC):
                w = _splat_f32(w_v, e)
                for d in range(SW // L):
                    sl = pl.ds(d * L, L)
                    rows_v[e, sl] = rows_v[e, sl] * w
            pltpu.sync_copy(rows_v, acc_sh.at[dst_v.at[0]], add=True)

        plsc.subcore_barrier()
        pltpu.sync_copy(
            acc_sh.at[pl.ds(r0, rows_per_tile), :],
            out_hbm.at[s, pl.ds(r0, rows_per_tile), :])
        plsc.subcore_barrier()


MB = 128


def _mean_body(a_ref, b_ref, c_ref, o_ref):
    scale = jnp.float32(1.0 / 3.0)
    for s in range(SLABS):
        o_ref[:, s * SW:(s + 1) * SW] = (
            a_ref[s] + b_ref[s] + c_ref[s]) * scale


_mean_call = pl.pallas_call(
    _mean_body,
    out_shape=jax.ShapeDtypeStruct((N, D), jnp.float32),
    grid=(pl.cdiv(N, MB),),
    in_specs=[pl.BlockSpec((SLABS, MB, SW), lambda i: (0, i, 0))] * 3,
    out_specs=pl.BlockSpec((MB, D), lambda i: (i, 0)),
)


def kernel(centroid_embs, finetune_embs, assign_val, ft_assign_val,
           edge_weight, assign_idx, ft_assign_idx, edge_index):
    ft_idx = ft_assign_idx.astype(jnp.int32).reshape(-1)
    ft_val = ft_assign_val.reshape(-1)
    refined = _refine_kernel(centroid_embs, finetune_embs, ft_val, ft_idx)

    idx = jnp.pad(assign_idx.astype(jnp.int32), ((0, N_PAD - N), (0, 0)))
    val = jnp.pad(assign_val, ((0, N_PAD - N), (0, 0)))
    x0s = _compose_kernel(refined, val.reshape(-1), idx.reshape(-1))

    src = jnp.pad(edge_index[0].astype(jnp.int32), (0, E_PAD - E))
    dst = jnp.pad(edge_index[1].astype(jnp.int32), (0, E_PAD - E))
    w = jnp.pad(edge_weight, (0, E_PAD - E))
    x1s = _prop_kernel(x0s.reshape(-1, SW), src, dst, w)
    x2s = _prop_kernel(x1s.reshape(-1, SW), src, dst, w)

    return _mean_call(x0s, x1s, x2s)


# speedup vs baseline: 2.9874x; 1.6658x over previous
import functools

import jax
import jax.numpy as jnp
from jax import lax
from jax.experimental import pallas as pl
from jax.experimental.pallas import tpu as pltpu
from jax.experimental.pallas import tpu_sc as plsc

NC = 2
NS = 16
NW = NC * NS
L = 16

N = 50000
D = 128
C1 = 8192
C2 = 1024
K1 = 8
K2 = 4
E = 800000

SLABS = 4
SW = D // SLABS
N_PAD = 50176
E_PAD = 802816

_mesh = plsc.VectorSubcoreMesh(core_axis_name="c", subcore_axis_name="s")


def _wid():
    return lax.axis_index("s") * NC + lax.axis_index("c")


def _splat_f32(ref, i):
    v = ref[pl.ds((i // L) * L, L)]
    idx = jnp.full((L,), i % L, dtype=jnp.int32)
    dnums = lax.GatherDimensionNumbers(
        offset_dims=(), collapsed_slice_dims=(0,), start_index_map=(0,))
    return lax.gather(v, idx[:, None], dnums, (1,),
                      mode=lax.GatherScatterMode.PROMISE_IN_BOUNDS)


RF_RC = 16
RF_STEPS = C1 // NW // RF_RC


@functools.partial(
    pl.kernel,
    out_type=jax.ShapeDtypeStruct((C1, D), jnp.float32),
    mesh=_mesh,
    scratch_types=[
        pltpu.VMEM((RF_RC, D), jnp.float32),
        pltpu.VMEM((RF_RC * K2,), jnp.int32),
        pltpu.VMEM((RF_RC * K2,), jnp.float32),
        pltpu.VMEM((RF_RC * K2, D), jnp.float32),
        pltpu.VMEM((RF_RC, D), jnp.float32),
        pltpu.SemaphoreType.DMA,
    ],
)
def _refine_kernel(cen_hbm, ft_hbm, val_hbm, idx_hbm, out_hbm,
                   cen_v, idx_v, val_v, rows_v, out_v, sem):
    base = _wid() * (RF_RC * RF_STEPS)

    @pl.loop(0, RF_STEPS)
    def _(t):
        off = base + t * RF_RC
        pltpu.sync_copy(cen_hbm.at[pl.ds(off, RF_RC), :], cen_v)
        pltpu.sync_copy(idx_hbm.at[pl.ds(off * K2, RF_RC * K2)], idx_v)
        pltpu.sync_copy(val_hbm.at[pl.ds(off * K2, RF_RC * K2)], val_v)
        pltpu.async_copy(ft_hbm.at[idx_v], rows_v, sem).wait()
        for r in range(RF_RC):
            w = [_splat_f32(val_v, r * K2 + k) for k in range(K2)]
            for d in range(D // L):
                sl = pl.ds(d * L, L)
                acc = cen_v[r, sl]
                for k in range(K2):
                    acc = acc + w[k] * rows_v[r * K2 + k, sl]
                out_v[r, sl] = acc
        pltpu.sync_copy(out_v, out_hbm.at[pl.ds(off, RF_RC), :])


X0_EC = 16
X0_STEPS = N_PAD // NW // X0_EC


@functools.partial(
    pl.kernel,
    out_type=jax.ShapeDtypeStruct((SLABS, N_PAD, SW), jnp.float32),
    mesh=_mesh,
    scratch_types=[
        pltpu.VMEM((X0_EC * K1,), jnp.int32),
        pltpu.VMEM((X0_EC * K1,), jnp.float32),
        pltpu.VMEM((X0_EC * K1, D), jnp.float32),
        pltpu.VMEM((SLABS, X0_EC, SW), jnp.float32),
        pltpu.SemaphoreType.DMA,
    ],
    compiler_params=pltpu.CompilerParams(use_tc_tiling_on_sc=False),
)
def _compose_kernel(ref_hbm, val_hbm, idx_hbm, out_hbm,
                    idx_v, val_v, rows_v, out_v, sem):
    base = _wid() * (X0_EC * X0_STEPS)

    @pl.loop(0, X0_STEPS)
    def _(t):
        off = base + t * X0_EC
        pltpu.sync_copy(idx_hbm.at[pl.ds(off * K1, X0_EC * K1)], idx_v)
        pltpu.sync_copy(val_hbm.at[pl.ds(off * K1, X0_EC * K1)], val_v)
        pltpu.async_copy(ref_hbm.at[idx_v], rows_v, sem).wait()
        for e in range(X0_EC):
            w = [_splat_f32(val_v, e * K1 + k) for k in range(K1)]
            for d in range(D // L):
                sl = pl.ds((d % 2) * L, L)
                acc = w[0] * rows_v[e * K1, pl.ds(d * L, L)]
                for k in range(1, K1):
                    acc = acc + w[k] * rows_v[e * K1 + k, pl.ds(d * L, L)]
                out_v[d // 2, e, sl] = acc
        for s in range(SLABS):
            pltpu.sync_copy(out_v.at[s],
                            out_hbm.at[s, pl.ds(off, X0_EC), :])


PR_EC = 128
PR_B = 4
PR_BODIES = 98
N_ACC = 50048
ZB_ROWS = 136


@functools.partial(
    pl.kernel,
    out_type=jax.ShapeDtypeStruct((SLABS, N_PAD, SW), jnp.float32),
    mesh=_mesh,
    scratch_types=[
        pltpu.VMEM((PR_B, PR_EC), jnp.int32),
        pltpu.VMEM((PR_B, PR_EC), jnp.int32),
        pltpu.VMEM((PR_B * PR_EC,), jnp.float32),
        pltpu.VMEM((PR_B, PR_EC), jnp.int32),
        pltpu.VMEM((PR_B, PR_EC, SW), jnp.float32),
        pltpu.VMEM((ZB_ROWS, SW), jnp.float32),
        pltpu.VMEM_SHARED((N_ACC, SW), jnp.float32),
        pltpu.SemaphoreType.DMA,
        pltpu.SemaphoreType.DMA,
        pltpu.SemaphoreType.DMA,
        pltpu.SemaphoreType.DMA,
        pltpu.SemaphoreType.DMA,
    ],
    compiler_params=pltpu.CompilerParams(use_tc_tiling_on_sc=False),
)
def _prop_kernel(x_hbm, src_hbm, dst_hbm, w_hbm, out_hbm,
                 ms, md, mw, gidx, rows, zb_v, acc_sh,
                 sg0, sg1, sg2, sg3, ss):
    c = lax.axis_index("c")
    sid = lax.axis_index("s")
    sems = [sg0, sg1, sg2, sg3]
    body_rows = PR_B * PR_EC // PR_EC
    ebase = sid * (PR_B * PR_BODIES)
    rows_per_tile = N_ACC // NS
    r0 = sid * rows_per_tile

    for i in range(ZB_ROWS):
        for j in range(SW // L):
            zb_v[i, pl.ds(j * L, L)] = jnp.zeros((L,), jnp.float32)

    for j in range(2):
        s = c * 2 + j
        for i in range(rows_per_tile // ZB_ROWS):
            pltpu.sync_copy(zb_v,
                            acc_sh.at[pl.ds(r0 + i * ZB_ROWS, ZB_ROWS), :])
        plsc.subcore_barrier()

        sbase = s * N_PAD

        @pl.loop(0, PR_BODIES)
        def _(t):
            roff = ebase + t * PR_B
            pltpu.sync_copy(src_hbm.at[pl.ds(roff, PR_B), :], ms)
            pltpu.sync_copy(dst_hbm.at[pl.ds(roff, PR_B), :], md)
            pltpu.sync_copy(w_hbm.at[pl.ds(roff * PR_EC, PR_B * PR_EC)], mw)
            sb = jnp.full((L,), sbase, dtype=jnp.int32)
            gd = []
            for i in range(PR_B):
                for k in range(PR_EC // L):
                    sl = pl.ds(k * L, L)
                    gidx[i, sl] = ms[i, sl] + sb
                gd.append(pltpu.async_copy(
                    x_hbm.at[gidx.at[i]], rows.at[i], sems[i]))
            sd = []
            for i in range(PR_B):
                gd[i].wait()
                for e in range(PR_EC):
                    w = _splat_f32(mw, i * PR_EC + e)
                    for d in range(SW // L):
                        sl = pl.ds(d * L, L)
                        rows[i, e, sl] = rows[i, e, sl] * w
                sd.append(pltpu.async_copy(
                    rows.at[i], acc_sh.at[md.at[i]], ss, add=True))
            for i in range(PR_B):
                sd[i].wait()

        plsc.subcore_barrier()
        pltpu.sync_copy(
            acc_sh.at[pl.ds(r0, rows_per_tile), :],
            out_hbm.at[s, pl.ds(r0, rows_per_tile), :])
        plsc.subcore_barrier()


MB = 128


def _mean_body(a_ref, b_ref, c_ref, o_ref):
    scale = jnp.float32(1.0 / 3.0)
    for s in range(SLABS):
        o_ref[:, s * SW:(s + 1) * SW] = (
            a_ref[s] + b_ref[s] + c_ref[s]) * scale


_mean_call = pl.pallas_call(
    _mean_body,
    out_shape=jax.ShapeDtypeStruct((N, D), jnp.float32),
    grid=(pl.cdiv(N, MB),),
    in_specs=[pl.BlockSpec((SLABS, MB, SW), lambda i: (0, i, 0))] * 3,
    out_specs=pl.BlockSpec((MB, D), lambda i: (i, 0)),
)


def kernel(centroid_embs, finetune_embs, assign_val, ft_assign_val,
           edge_weight, assign_idx, ft_assign_idx, edge_index):
    ft_idx = ft_assign_idx.astype(jnp.int32).reshape(-1)
    ft_val = ft_assign_val.reshape(-1)
    refined = _refine_kernel(centroid_embs, finetune_embs, ft_val, ft_idx)

    idx = jnp.pad(assign_idx.astype(jnp.int32), ((0, N_PAD - N), (0, 0)))
    val = jnp.pad(assign_val, ((0, N_PAD - N), (0, 0)))
    x0s = _compose_kernel(refined, val.reshape(-1), idx.reshape(-1))

    src = jnp.pad(edge_index[0].astype(jnp.int32),
                  (0, E_PAD - E)).reshape(-1, PR_EC)
    dst = jnp.pad(edge_index[1].astype(jnp.int32),
                  (0, E_PAD - E)).reshape(-1, PR_EC)
    w = jnp.pad(edge_weight, (0, E_PAD - E))
    x1s = _prop_kernel(x0s.reshape(-1, SW), src, dst, w)
    x2s = _prop_kernel(x1s.reshape(-1, SW), src, dst, w)

    return _mean_call(x0s, x1s, x2s)


# speedup vs baseline: 3.7629x; 1.2596x over previous
import functools

import jax
import jax.numpy as jnp
from jax import lax
from jax.experimental import pallas as pl
from jax.experimental.pallas import tpu as pltpu
from jax.experimental.pallas import tpu_sc as plsc

NC = 2
NS = 16
NW = NC * NS
L = 16

N = 50000
D = 128
C1 = 8192
C2 = 1024
K1 = 8
K2 = 4
E = 800000

SLABS = 4
SW = D // SLABS
N_PAD = 50176
E_PAD = 802816

_mesh = plsc.VectorSubcoreMesh(core_axis_name="c", subcore_axis_name="s")


def _wid():
    return lax.axis_index("s") * NC + lax.axis_index("c")


def _lane_splat(v, lane):
    idx = jnp.full((L,), lane, dtype=jnp.int32)
    dnums = lax.GatherDimensionNumbers(
        offset_dims=(), collapsed_slice_dims=(0,), start_index_map=(0,))
    return lax.gather(v, idx[:, None], dnums, (1,),
                      mode=lax.GatherScatterMode.PROMISE_IN_BOUNDS)


def _splat_f32(ref, i):
    return _lane_splat(ref[pl.ds((i // L) * L, L)], i % L)


RF_RC = 16
RF_STEPS = C1 // NW // RF_RC


@functools.partial(
    pl.kernel,
    out_type=jax.ShapeDtypeStruct((C1, D), jnp.float32),
    mesh=_mesh,
    scratch_types=[
        pltpu.VMEM((RF_RC, D), jnp.float32),
        pltpu.VMEM((RF_RC * K2,), jnp.int32),
        pltpu.VMEM((RF_RC * K2,), jnp.float32),
        pltpu.VMEM((RF_RC * K2, D), jnp.float32),
        pltpu.VMEM((RF_RC, D), jnp.float32),
        pltpu.SemaphoreType.DMA,
    ],
)
def _refine_kernel(cen_hbm, ft_hbm, val_hbm, idx_hbm, out_hbm,
                   cen_v, idx_v, val_v, rows_v, out_v, sem):
    base = _wid() * (RF_RC * RF_STEPS)

    @pl.loop(0, RF_STEPS)
    def _(t):
        off = base + t * RF_RC
        pltpu.sync_copy(cen_hbm.at[pl.ds(off, RF_RC), :], cen_v)
        pltpu.sync_copy(idx_hbm.at[pl.ds(off * K2, RF_RC * K2)], idx_v)
        pltpu.sync_copy(val_hbm.at[pl.ds(off * K2, RF_RC * K2)], val_v)
        pltpu.async_copy(ft_hbm.at[idx_v], rows_v, sem).wait()
        for r in range(RF_RC):
            w = [_splat_f32(val_v, r * K2 + k) for k in range(K2)]
            for d in range(D // L):
                sl = pl.ds(d * L, L)
                acc = cen_v[r, sl]
                for k in range(K2):
                    acc = acc + w[k] * rows_v[r * K2 + k, sl]
                out_v[r, sl] = acc
        pltpu.sync_copy(out_v, out_hbm.at[pl.ds(off, RF_RC), :])


X0_EC = 8
X0_B = 4
X0_BODIES = N_PAD // NW // (X0_EC * X0_B)


@functools.partial(
    pl.kernel,
    out_type=jax.ShapeDtypeStruct((SLABS, N_PAD, SW), jnp.float32),
    mesh=_mesh,
    scratch_types=[
        pltpu.VMEM((X0_B, X0_EC * K1), jnp.int32),
        pltpu.VMEM((X0_B * X0_EC * K1,), jnp.float32),
        pltpu.VMEM((X0_B, X0_EC * K1, D), jnp.float32),
        pltpu.VMEM((X0_B, SLABS, X0_EC, SW), jnp.float32),
        pltpu.SemaphoreType.DMA,
        pltpu.SemaphoreType.DMA,
        pltpu.SemaphoreType.DMA,
        pltpu.SemaphoreType.DMA,
        pltpu.SemaphoreType.DMA,
    ],
    compiler_params=pltpu.CompilerParams(use_tc_tiling_on_sc=False),
)
def _compose_kernel(ref_hbm, val_hbm, idx_hbm, out_hbm,
                    idx_v, val_v, rows, out_v, sg0, sg1, sg2, sg3, so):
    base = _wid() * (X0_EC * X0_B * X0_BODIES)
    sems = [sg0, sg1, sg2, sg3]

    @pl.loop(0, X0_BODIES)
    def _(t):
        off = base + t * (X0_EC * X0_B)
        pltpu.sync_copy(
            idx_hbm.at[pl.ds(_wid() * (X0_BODIES * X0_B) + t * X0_B,
                             X0_B), :], idx_v)
        pltpu.sync_copy(
            val_hbm.at[pl.ds(off * K1, X0_B * X0_EC * K1)], val_v)
        gd = []
        for i in range(X0_B):
            gd.append(pltpu.async_copy(
                ref_hbm.at[idx_v.at[i]], rows.at[i], sems[i]))
        sd = []
        for i in range(X0_B):
            gd[i].wait()
            for e in range(X0_EC):
                vbase = i * X0_EC * K1 + e * K1
                v16 = val_v[pl.ds((vbase // L) * L, L)]
                lb = vbase % L
                w = [_lane_splat(v16, lb + k) for k in range(K1)]
                for d in range(D // L):
                    sl = pl.ds((d % 2) * L, L)
                    acc = w[0] * rows[i, e * K1, pl.ds(d * L, L)]
                    for k in range(1, K1):
                        acc = acc + w[k] * rows[i, e * K1 + k, pl.ds(d * L, L)]
                    out_v[i, d // 2, e, sl] = acc
            for s in range(SLABS):
                sd.append(pltpu.async_copy(
                    out_v.at[i, s],
                    out_hbm.at[s, pl.ds(off + i * X0_EC, X0_EC), :], so))
        for d in sd:
            d.wait()


PR_EC = 128
PR_B = 4
PR_BODIES = 98
N_ACC = 50048
ZB_ROWS = 136


@functools.partial(
    pl.kernel,
    out_type=jax.ShapeDtypeStruct((SLABS, N_PAD, SW), jnp.float32),
    mesh=_mesh,
    scratch_types=[
        pltpu.VMEM((PR_B, PR_EC), jnp.int32),
        pltpu.VMEM((PR_B, PR_EC), jnp.int32),
        pltpu.VMEM((PR_B * PR_EC,), jnp.float32),
        pltpu.VMEM((PR_B, PR_EC), jnp.int32),
        pltpu.VMEM((PR_B, PR_EC, SW), jnp.float32),
        pltpu.VMEM((ZB_ROWS, SW), jnp.float32),
        pltpu.VMEM_SHARED((N_ACC, SW), jnp.float32),
        pltpu.SemaphoreType.DMA,
        pltpu.SemaphoreType.DMA,
        pltpu.SemaphoreType.DMA,
        pltpu.SemaphoreType.DMA,
        pltpu.SemaphoreType.DMA,
    ],
    compiler_params=pltpu.CompilerParams(use_tc_tiling_on_sc=False),
)
def _prop_kernel(x_hbm, src_hbm, dst_hbm, w_hbm, out_hbm,
                 ms, md, mw, gidx, rows, zb_v, acc_sh,
                 sg0, sg1, sg2, sg3, ss):
    c = lax.axis_index("c")
    sid = lax.axis_index("s")
    sems = [sg0, sg1, sg2, sg3]
    body_rows = PR_B * PR_EC // PR_EC
    ebase = sid * (PR_B * PR_BODIES)
    rows_per_tile = N_ACC // NS
    r0 = sid * rows_per_tile

    for i in range(ZB_ROWS):
        for j in range(SW // L):
            zb_v[i, pl.ds(j * L, L)] = jnp.zeros((L,), jnp.float32)

    for j in range(2):
        s = c * 2 + j
        for i in range(rows_per_tile // ZB_ROWS):
            pltpu.sync_copy(zb_v,
                            acc_sh.at[pl.ds(r0 + i * ZB_ROWS, ZB_ROWS), :])
        plsc.subcore_barrier()

        sbase = s * N_PAD

        @pl.loop(0, PR_BODIES)
        def _(t):
            roff = ebase + t * PR_B
            pltpu.sync_copy(src_hbm.at[pl.ds(roff, PR_B), :], ms)
            pltpu.sync_copy(dst_hbm.at[pl.ds(roff, PR_B), :], md)
            pltpu.sync_copy(w_hbm.at[pl.ds(roff * PR_EC, PR_B * PR_EC)], mw)
            sb = jnp.full((L,), sbase, dtype=jnp.int32)
            gd = []
            for i in range(PR_B):
                for k in range(PR_EC // L):
                    sl = pl.ds(k * L, L)
                    gidx[i, sl] = ms[i, sl] + sb
                gd.append(pltpu.async_copy(
                    x_hbm.at[gidx.at[i]], rows.at[i], sems[i]))
            sd = []
            for i in range(PR_B):
                gd[i].wait()
                for g in range(PR_EC // L):
                    w16 = mw[pl.ds((i * PR_EC // L + g) * L, L)]
                    for el in range(L):
                        e = g * L + el
                        w = _lane_splat(w16, el)
                        for d in range(SW // L):
                            sl = pl.ds(d * L, L)
                            rows[i, e, sl] = rows[i, e, sl] * w
                sd.append(pltpu.async_copy(
                    rows.at[i], acc_sh.at[md.at[i]], ss, add=True))
            for i in range(PR_B):
                sd[i].wait()

        plsc.subcore_barrier()
        pltpu.sync_copy(
            acc_sh.at[pl.ds(r0, rows_per_tile), :],
            out_hbm.at[s, pl.ds(r0, rows_per_tile), :])
        plsc.subcore_barrier()


MB = 128


def _mean_body(a_ref, b_ref, c_ref, o_ref):
    scale = jnp.float32(1.0 / 3.0)
    for s in range(SLABS):
        o_ref[:, s * SW:(s + 1) * SW] = (
            a_ref[s] + b_ref[s] + c_ref[s]) * scale


_mean_call = pl.pallas_call(
    _mean_body,
    out_shape=jax.ShapeDtypeStruct((N, D), jnp.float32),
    grid=(pl.cdiv(N, MB),),
    in_specs=[pl.BlockSpec((SLABS, MB, SW), lambda i: (0, i, 0))] * 3,
    out_specs=pl.BlockSpec((MB, D), lambda i: (i, 0)),
)


def kernel(centroid_embs, finetune_embs, assign_val, ft_assign_val,
           edge_weight, assign_idx, ft_assign_idx, edge_index):
    ft_idx = ft_assign_idx.astype(jnp.int32).reshape(-1)
    ft_val = ft_assign_val.reshape(-1)
    refined = _refine_kernel(centroid_embs, finetune_embs, ft_val, ft_idx)

    idx = jnp.pad(assign_idx.astype(jnp.int32), ((0, N_PAD - N), (0, 0)))
    val = jnp.pad(assign_val, ((0, N_PAD - N), (0, 0)))
    x0s = _compose_kernel(refined, val.reshape(-1),
                          idx.reshape(-1, X0_EC * K1))

    src = jnp.pad(edge_index[0].astype(jnp.int32),
                  (0, E_PAD - E)).reshape(-1, PR_EC)
    dst = jnp.pad(edge_index[1].astype(jnp.int32),
                  (0, E_PAD - E)).reshape(-1, PR_EC)
    w = jnp.pad(edge_weight, (0, E_PAD - E))
    x1s = _prop_kernel(x0s.reshape(-1, SW), src, dst, w)
    x2s = _prop_kernel(x1s.reshape(-1, SW), src, dst, w)

    return _mean_call(x0s, x1s, x2s)


# speedup vs baseline: 4.2178x; 1.1209x over previous
import functools

import jax
import jax.numpy as jnp
from jax import lax
from jax.experimental import pallas as pl
from jax.experimental.pallas import tpu as pltpu
from jax.experimental.pallas import tpu_sc as plsc

NC = 2
NS = 16
NW = NC * NS
L = 16

N = 50000
D = 128
C1 = 8192
C2 = 1024
K1 = 8
K2 = 4
E = 800000

SLABS = 4
SW = D // SLABS
N_PAD = 50176
E_PAD = 802816

_mesh = plsc.VectorSubcoreMesh(core_axis_name="c", subcore_axis_name="s")


def _wid():
    return lax.axis_index("s") * NC + lax.axis_index("c")


def _lane_splat(v, lane):
    idx = jnp.full((L,), lane, dtype=jnp.int32)
    dnums = lax.GatherDimensionNumbers(
        offset_dims=(), collapsed_slice_dims=(0,), start_index_map=(0,))
    return lax.gather(v, idx[:, None], dnums, (1,),
                      mode=lax.GatherScatterMode.PROMISE_IN_BOUNDS)


def _splat_f32(ref, i):
    return _lane_splat(ref[pl.ds((i // L) * L, L)], i % L)


RF_RC = 16
RF_STEPS = C1 // NW // RF_RC


@functools.partial(
    pl.kernel,
    out_type=jax.ShapeDtypeStruct((C1, D), jnp.float32),
    mesh=_mesh,
    scratch_types=[
        pltpu.VMEM((RF_RC, D), jnp.float32),
        pltpu.VMEM((RF_RC * K2,), jnp.int32),
        pltpu.VMEM((RF_RC * K2,), jnp.float32),
        pltpu.VMEM((RF_RC * K2, D), jnp.float32),
        pltpu.VMEM((RF_RC, D), jnp.float32),
        pltpu.SemaphoreType.DMA,
    ],
)
def _refine_kernel(cen_hbm, ft_hbm, val_hbm, idx_hbm, out_hbm,
                   cen_v, idx_v, val_v, rows_v, out_v, sem):
    base = _wid() * (RF_RC * RF_STEPS)

    @pl.loop(0, RF_STEPS)
    def _(t):
        off = base + t * RF_RC
        pltpu.sync_copy(cen_hbm.at[pl.ds(off, RF_RC), :], cen_v)
        pltpu.sync_copy(idx_hbm.at[pl.ds(off * K2, RF_RC * K2)], idx_v)
        pltpu.sync_copy(val_hbm.at[pl.ds(off * K2, RF_RC * K2)], val_v)
        pltpu.async_copy(ft_hbm.at[idx_v], rows_v, sem).wait()
        for r in range(RF_RC):
            w = [_splat_f32(val_v, r * K2 + k) for k in range(K2)]
            for d in range(D // L):
                sl = pl.ds(d * L, L)
                acc = cen_v[r, sl]
                for k in range(K2):
                    acc = acc + w[k] * rows_v[r * K2 + k, sl]
                out_v[r, sl] = acc
        pltpu.sync_copy(out_v, out_hbm.at[pl.ds(off, RF_RC), :])


X0_EC = 8
X0_B = 4
X0_BODIES = N_PAD // NW // (X0_EC * X0_B)


@functools.partial(
    pl.kernel,
    out_type=jax.ShapeDtypeStruct((SLABS, N_PAD, SW), jnp.float32),
    mesh=_mesh,
    scratch_types=[
        pltpu.VMEM((X0_B, X0_EC * K1), jnp.int32),
        pltpu.VMEM((X0_B * X0_EC * K1,), jnp.float32),
        pltpu.VMEM((X0_B, X0_EC * K1, D), jnp.float32),
        pltpu.VMEM((X0_B, SLABS, X0_EC, SW), jnp.float32),
        pltpu.SemaphoreType.DMA,
        pltpu.SemaphoreType.DMA,
        pltpu.SemaphoreType.DMA,
        pltpu.SemaphoreType.DMA,
        pltpu.SemaphoreType.DMA,
    ],
    compiler_params=pltpu.CompilerParams(use_tc_tiling_on_sc=False),
)
def _compose_kernel(ref_hbm, val_hbm, idx_hbm, out_hbm,
                    idx_v, val_v, rows, out_v, sg0, sg1, sg2, sg3, so):
    base = _wid() * (X0_EC * X0_B * X0_BODIES)
    sems = [sg0, sg1, sg2, sg3]

    @pl.loop(0, X0_BODIES)
    def _(t):
        off = base + t * (X0_EC * X0_B)
        pltpu.sync_copy(
            idx_hbm.at[pl.ds(_wid() * (X0_BODIES * X0_B) + t * X0_B,
                             X0_B), :], idx_v)
        pltpu.sync_copy(
            val_hbm.at[pl.ds(off * K1, X0_B * X0_EC * K1)], val_v)
        gd = []
        for i in range(X0_B):
            gd.append(pltpu.async_copy(
                ref_hbm.at[idx_v.at[i]], rows.at[i], sems[i]))
        sd = []
        for i in range(X0_B):
            gd[i].wait()
            for e in range(X0_EC):
                vbase = i * X0_EC * K1 + e * K1
                v16 = val_v[pl.ds((vbase // L) * L, L)]
                lb = vbase % L
                w = [_lane_splat(v16, lb + k) for k in range(K1)]
                for d in range(D // L):
                    sl = pl.ds((d % 2) * L, L)
                    acc = w[0] * rows[i, e * K1, pl.ds(d * L, L)]
                    for k in range(1, K1):
                        acc = acc + w[k] * rows[i, e * K1 + k, pl.ds(d * L, L)]
                    out_v[i, d // 2, e, sl] = acc
            for s in range(SLABS):
                sd.append(pltpu.async_copy(
                    out_v.at[i, s],
                    out_hbm.at[s, pl.ds(off + i * X0_EC, X0_EC), :], so))
        for d in sd:
            d.wait()


PR_EC = 128
PR_B = 4
PR_HB = PR_B * PR_EC
PR_BODIES = 49
N_ACC = 50048
ZB_ROWS = 136


def _meta_load(src_hbm, dst_hbm, w_hbm, roff, ms, md, mw, sem):
    d = [pltpu.async_copy(src_hbm.at[pl.ds(roff, PR_B), :], ms, sem),
         pltpu.async_copy(dst_hbm.at[pl.ds(roff, PR_B), :], md, sem),
         pltpu.async_copy(w_hbm.at[pl.ds(roff * PR_EC, PR_HB)], mw, sem)]
    return d


@functools.partial(
    pl.kernel,
    out_type=jax.ShapeDtypeStruct((SLABS, N_PAD, SW), jnp.float32),
    mesh=_mesh,
    scratch_types=[
        pltpu.VMEM((PR_B, PR_EC), jnp.int32),
        pltpu.VMEM((PR_B, PR_EC), jnp.int32),
        pltpu.VMEM((PR_HB,), jnp.float32),
        pltpu.VMEM((PR_B, PR_EC), jnp.int32),
        pltpu.VMEM((PR_B, PR_EC), jnp.int32),
        pltpu.VMEM((PR_HB,), jnp.float32),
        pltpu.VMEM((PR_B, PR_EC, SW), jnp.float32),
        pltpu.VMEM((ZB_ROWS, SW), jnp.float32),
        pltpu.VMEM_SHARED((N_ACC, SW), jnp.float32),
        pltpu.SemaphoreType.DMA,
        pltpu.SemaphoreType.DMA,
        pltpu.SemaphoreType.DMA,
        pltpu.SemaphoreType.DMA,
        pltpu.SemaphoreType.DMA,
        pltpu.SemaphoreType.DMA,
    ],
    compiler_params=pltpu.CompilerParams(use_tc_tiling_on_sc=False),
)
def _prop_kernel(x_hbm, src_hbm, dst_hbm, w_hbm, out_hbm,
                 msa, mda, mwa, msb, mdb, mwb, rows, zb_v, acc_sh,
                 sg0, sg1, sg2, sg3, ss, sm):
    c = lax.axis_index("c")
    sid = lax.axis_index("s")
    sems = [sg0, sg1, sg2, sg3]
    ebase = sid * (2 * PR_B * PR_BODIES)
    rows_per_tile = N_ACC // NS
    r0 = sid * rows_per_tile

    for i in range(ZB_ROWS):
        for j in range(SW // L):
            zb_v[i, pl.ds(j * L, L)] = jnp.zeros((L,), jnp.float32)

    def half(roff, ms, md, mw, sbase, prefetch):
        sb = jnp.full((L,), sbase, dtype=jnp.int32)
        gd = []
        for i in range(PR_B):
            for k in range(PR_EC // L):
                sl = pl.ds(k * L, L)
                ms[i, sl] = ms[i, sl] + sb
            gd.append(pltpu.async_copy(
                x_hbm.at[ms.at[i]], rows.at[i], sems[i]))
        pf = prefetch()
        sd = []
        for i in range(PR_B):
            gd[i].wait()
            for g in range(PR_EC // L):
                w16 = mw[pl.ds(g * L + i * PR_EC, L)]
                for el in range(L):
                    e = g * L + el
                    w = _lane_splat(w16, el)
                    for d in range(SW // L):
                        sl = pl.ds(d * L, L)
                        rows[i, e, sl] = rows[i, e, sl] * w
            sd.append(pltpu.async_copy(
                rows.at[i], acc_sh.at[md.at[i]], ss, add=True))
        for d in sd:
            d.wait()
        for d in pf:
            d.wait()

    for j in range(2):
        s = c * 2 + j
        for i in range(rows_per_tile // ZB_ROWS):
            pltpu.sync_copy(zb_v,
                            acc_sh.at[pl.ds(r0 + i * ZB_ROWS, ZB_ROWS), :])
        plsc.subcore_barrier()

        sbase = s * N_PAD
        for d in _meta_load(src_hbm, dst_hbm, w_hbm, ebase, msa, mda, mwa,
                            sm):
            d.wait()

        @pl.loop(0, PR_BODIES)
        def _(t):
            roff = ebase + t * (2 * PR_B)
            half(roff, msa, mda, mwa, sbase,
                 lambda: _meta_load(src_hbm, dst_hbm, w_hbm, roff + PR_B,
                                    msb, mdb, mwb, sm))
            nxt = jnp.where(t + 1 < PR_BODIES, roff + 2 * PR_B, ebase)
            half(roff + PR_B, msb, mdb, mwb, sbase,
                 lambda: _meta_load(src_hbm, dst_hbm, w_hbm, nxt,
                                    msa, mda, mwa, sm))

        plsc.subcore_barrier()
        pltpu.sync_copy(
            acc_sh.at[pl.ds(r0, rows_per_tile), :],
            out_hbm.at[s, pl.ds(r0, rows_per_tile), :])
        plsc.subcore_barrier()


MB = 128


def _mean_body(a_ref, b_ref, c_ref, o_ref):
    scale = jnp.float32(1.0 / 3.0)
    for s in range(SLABS):
        o_ref[:, s * SW:(s + 1) * SW] = (
            a_ref[s] + b_ref[s] + c_ref[s]) * scale


_mean_call = pl.pallas_call(
    _mean_body,
    out_shape=jax.ShapeDtypeStruct((N, D), jnp.float32),
    grid=(pl.cdiv(N, MB),),
    in_specs=[pl.BlockSpec((SLABS, MB, SW), lambda i: (0, i, 0))] * 3,
    out_specs=pl.BlockSpec((MB, D), lambda i: (i, 0)),
)


def kernel(centroid_embs, finetune_embs, assign_val, ft_assign_val,
           edge_weight, assign_idx, ft_assign_idx, edge_index):
    ft_idx = ft_assign_idx.astype(jnp.int32).reshape(-1)
    ft_val = ft_assign_val.reshape(-1)
    refined = _refine_kernel(centroid_embs, finetune_embs, ft_val, ft_idx)

    idx = jnp.pad(assign_idx.astype(jnp.int32), ((0, N_PAD - N), (0, 0)))
    val = jnp.pad(assign_val, ((0, N_PAD - N), (0, 0)))
    x0s = _compose_kernel(refined, val.reshape(-1),
                          idx.reshape(-1, X0_EC * K1))

    src = jnp.pad(edge_index[0].astype(jnp.int32),
                  (0, E_PAD - E)).reshape(-1, PR_EC)
    dst = jnp.pad(edge_index[1].astype(jnp.int32),
                  (0, E_PAD - E)).reshape(-1, PR_EC)
    w = jnp.pad(edge_weight, (0, E_PAD - E))
    x1s = _prop_kernel(x0s.reshape(-1, SW), src, dst, w)
    x2s = _prop_kernel(x1s.reshape(-1, SW), src, dst, w)

    return _mean_call(x0s, x1s, x2s)


# speedup vs baseline: 4.2837x; 1.0156x over previous
import functools

import jax
import jax.numpy as jnp
from jax import lax
from jax.experimental import pallas as pl
from jax.experimental.pallas import tpu as pltpu
from jax.experimental.pallas import tpu_sc as plsc

NC = 2
NS = 16
NW = NC * NS
L = 16

N = 50000
D = 128
C1 = 8192
C2 = 1024
K1 = 8
K2 = 4
E = 800000

SLABS = 4
SW = D // SLABS
N_PAD = 50176
E_PAD = 802816

_mesh = plsc.VectorSubcoreMesh(core_axis_name="c", subcore_axis_name="s")


def _wid():
    return lax.axis_index("s") * NC + lax.axis_index("c")


def _lane_splat(v, lane):
    idx = jnp.full((L,), lane, dtype=jnp.int32)
    dnums = lax.GatherDimensionNumbers(
        offset_dims=(), collapsed_slice_dims=(0,), start_index_map=(0,))
    return lax.gather(v, idx[:, None], dnums, (1,),
                      mode=lax.GatherScatterMode.PROMISE_IN_BOUNDS)


def _splat_f32(ref, i):
    return _lane_splat(ref[pl.ds((i // L) * L, L)], i % L)


RF_RC = 16
RF_STEPS = C1 // NW // RF_RC


@functools.partial(
    pl.kernel,
    out_type=jax.ShapeDtypeStruct((C1, D), jnp.float32),
    mesh=_mesh,
    scratch_types=[
        pltpu.VMEM((RF_RC, D), jnp.float32),
        pltpu.VMEM((RF_RC * K2,), jnp.int32),
        pltpu.VMEM((RF_RC * K2,), jnp.float32),
        pltpu.VMEM((RF_RC * K2, D), jnp.float32),
        pltpu.VMEM((RF_RC, D), jnp.float32),
        pltpu.SemaphoreType.DMA,
    ],
)
def _refine_kernel(cen_hbm, ft_hbm, val_hbm, idx_hbm, out_hbm,
                   cen_v, idx_v, val_v, rows_v, out_v, sem):
    base = _wid() * (RF_RC * RF_STEPS)

    @pl.loop(0, RF_STEPS)
    def _(t):
        off = base + t * RF_RC
        pltpu.sync_copy(cen_hbm.at[pl.ds(off, RF_RC), :], cen_v)
        pltpu.sync_copy(idx_hbm.at[pl.ds(off * K2, RF_RC * K2)], idx_v)
        pltpu.sync_copy(val_hbm.at[pl.ds(off * K2, RF_RC * K2)], val_v)
        pltpu.async_copy(ft_hbm.at[idx_v], rows_v, sem).wait()
        for r in range(RF_RC):
            w = [_splat_f32(val_v, r * K2 + k) for k in range(K2)]
            for d in range(D // L):
                sl = pl.ds(d * L, L)
                acc = cen_v[r, sl]
                for k in range(K2):
                    acc = acc + w[k] * rows_v[r * K2 + k, sl]
                out_v[r, sl] = acc
        pltpu.sync_copy(out_v, out_hbm.at[pl.ds(off, RF_RC), :])


X0_EC = 8
X0_B = 4
X0_BODIES = N_PAD // NW // (X0_EC * X0_B)


@functools.partial(
    pl.kernel,
    out_type=jax.ShapeDtypeStruct((SLABS, N_PAD, SW), jnp.float32),
    mesh=_mesh,
    scratch_types=[
        pltpu.VMEM((X0_B, X0_EC * K1), jnp.int32),
        pltpu.VMEM((X0_B * X0_EC * K1,), jnp.float32),
        pltpu.VMEM((X0_B, X0_EC * K1, D), jnp.float32),
        pltpu.VMEM((X0_B, SLABS, X0_EC, SW), jnp.float32),
        pltpu.SemaphoreType.DMA,
        pltpu.SemaphoreType.DMA,
        pltpu.SemaphoreType.DMA,
        pltpu.SemaphoreType.DMA,
        pltpu.SemaphoreType.DMA,
    ],
    compiler_params=pltpu.CompilerParams(use_tc_tiling_on_sc=False),
)
def _compose_kernel(ref_hbm, val_hbm, idx_hbm, out_hbm,
                    idx_v, val_v, rows, out_v, sg0, sg1, sg2, sg3, so):
    base = _wid() * (X0_EC * X0_B * X0_BODIES)
    sems = [sg0, sg1, sg2, sg3]

    @pl.loop(0, X0_BODIES)
    def _(t):
        off = base + t * (X0_EC * X0_B)
        pltpu.sync_copy(
            idx_hbm.at[pl.ds(_wid() * (X0_BODIES * X0_B) + t * X0_B,
                             X0_B), :], idx_v)
        pltpu.sync_copy(
            val_hbm.at[pl.ds(off * K1, X0_B * X0_EC * K1)], val_v)
        gd = []
        for i in range(X0_B):
            gd.append(pltpu.async_copy(
                ref_hbm.at[idx_v.at[i]], rows.at[i], sems[i]))
        sd = []
        for i in range(X0_B):
            gd[i].wait()
            for e in range(X0_EC):
                vbase = i * X0_EC * K1 + e * K1
                v16 = val_v[pl.ds((vbase // L) * L, L)]
                lb = vbase % L
                w = [_lane_splat(v16, lb + k) for k in range(K1)]
                for d in range(D // L):
                    sl = pl.ds((d % 2) * L, L)
                    dsl = pl.ds(d * L, L)
                    a0 = w[0] * rows[i, e * K1 + 0, dsl]
                    a1 = w[1] * rows[i, e * K1 + 1, dsl]
                    a2 = w[2] * rows[i, e * K1 + 2, dsl]
                    a3 = w[3] * rows[i, e * K1 + 3, dsl]
                    a0 = a0 + w[4] * rows[i, e * K1 + 4, dsl]
                    a1 = a1 + w[5] * rows[i, e * K1 + 5, dsl]
                    a2 = a2 + w[6] * rows[i, e * K1 + 6, dsl]
                    a3 = a3 + w[7] * rows[i, e * K1 + 7, dsl]
                    out_v[i, d // 2, e, sl] = (a0 + a1) + (a2 + a3)
            for s in range(SLABS):
                sd.append(pltpu.async_copy(
                    out_v.at[i, s],
                    out_hbm.at[s, pl.ds(off + i * X0_EC, X0_EC), :], so))
        for d in sd:
            d.wait()


PR_EC = 128
PR_B = 4
PR_HB = PR_B * PR_EC
PR_BODIES = 49
N_ACC = 50048
ZB_ROWS = 136


def _meta_load(src_hbm, dst_hbm, w_hbm, roff, ms, md, mw, sem):
    d = [pltpu.async_copy(src_hbm.at[pl.ds(roff, PR_B), :], ms, sem),
         pltpu.async_copy(dst_hbm.at[pl.ds(roff, PR_B), :], md, sem),
         pltpu.async_copy(w_hbm.at[pl.ds(roff * PR_EC, PR_HB)], mw, sem)]
    return d


@functools.partial(
    pl.kernel,
    out_type=jax.ShapeDtypeStruct((SLABS, N_PAD, SW), jnp.float32),
    mesh=_mesh,
    scratch_types=[
        pltpu.VMEM((PR_B, PR_EC), jnp.int32),
        pltpu.VMEM((PR_B, PR_EC), jnp.int32),
        pltpu.VMEM((PR_HB,), jnp.float32),
        pltpu.VMEM((PR_B, PR_EC), jnp.int32),
        pltpu.VMEM((PR_B, PR_EC), jnp.int32),
        pltpu.VMEM((PR_HB,), jnp.float32),
        pltpu.VMEM((PR_B, PR_EC, SW), jnp.float32),
        pltpu.VMEM((ZB_ROWS, SW), jnp.float32),
        pltpu.VMEM_SHARED((N_ACC, SW), jnp.float32),
        pltpu.SemaphoreType.DMA,
        pltpu.SemaphoreType.DMA,
        pltpu.SemaphoreType.DMA,
        pltpu.SemaphoreType.DMA,
        pltpu.SemaphoreType.DMA,
        pltpu.SemaphoreType.DMA,
    ],
    compiler_params=pltpu.CompilerParams(use_tc_tiling_on_sc=False),
)
def _prop_kernel(x_hbm, src_hbm, dst_hbm, w_hbm, out_hbm,
                 msa, mda, mwa, msb, mdb, mwb, rows, zb_v, acc_sh,
                 sg0, sg1, sg2, sg3, ss, sm):
    c = lax.axis_index("c")
    sid = lax.axis_index("s")
    sems = [sg0, sg1, sg2, sg3]
    ebase = sid * (2 * PR_B * PR_BODIES)
    rows_per_tile = N_ACC // NS
    r0 = sid * rows_per_tile

    for i in range(ZB_ROWS):
        for j in range(SW // L):
            zb_v[i, pl.ds(j * L, L)] = jnp.zeros((L,), jnp.float32)

    def half(roff, ms, md, mw, sbase, prefetch):
        sb = jnp.full((L,), sbase, dtype=jnp.int32)
        gd = []
        for i in range(PR_B):
            for k in range(PR_EC // L):
                sl = pl.ds(k * L, L)
                ms[i, sl] = ms[i, sl] + sb
            gd.append(pltpu.async_copy(
                x_hbm.at[ms.at[i]], rows.at[i], sems[i]))
        pf = prefetch()
        sd = []
        for i in range(PR_B):
            gd[i].wait()
            for g in range(PR_EC // L):
                w16 = mw[pl.ds(g * L + i * PR_EC, L)]
                for el in range(L):
                    e = g * L + el
                    w = _lane_splat(w16, el)
                    for d in range(SW // L):
                        sl = pl.ds(d * L, L)
                        rows[i, e, sl] = rows[i, e, sl] * w
            sd.append(pltpu.async_copy(
                rows.at[i], acc_sh.at[md.at[i]], ss, add=True))
        for d in sd:
            d.wait()
        for d in pf:
            d.wait()

    for j in range(2):
        s = c * 2 + j
        for i in range(rows_per_tile // ZB_ROWS):
            pltpu.sync_copy(zb_v,
                            acc_sh.at[pl.ds(r0 + i * ZB_ROWS, ZB_ROWS), :])
        plsc.subcore_barrier()

        sbase = s * N_PAD
        for d in _meta_load(src_hbm, dst_hbm, w_hbm, ebase, msa, mda, mwa,
                            sm):
            d.wait()

        @pl.loop(0, PR_BODIES)
        def _(t):
            roff = ebase + t * (2 * PR_B)
            half(roff, msa, mda, mwa, sbase,
                 lambda: _meta_load(src_hbm, dst_hbm, w_hbm, roff + PR_B,
                                    msb, mdb, mwb, sm))
            nxt = jnp.where(t + 1 < PR_BODIES, roff + 2 * PR_B, ebase)
            half(roff + PR_B, msb, mdb, mwb, sbase,
                 lambda: _meta_load(src_hbm, dst_hbm, w_hbm, nxt,
                                    msa, mda, mwa, sm))

        plsc.subcore_barrier()
        pltpu.sync_copy(
            acc_sh.at[pl.ds(r0, rows_per_tile), :],
            out_hbm.at[s, pl.ds(r0, rows_per_tile), :])
        plsc.subcore_barrier()


MB = 128


def _mean_body(a_ref, b_ref, c_ref, o_ref):
    scale = jnp.float32(1.0 / 3.0)
    for s in range(SLABS):
        o_ref[:, s * SW:(s + 1) * SW] = (
            a_ref[s] + b_ref[s] + c_ref[s]) * scale


_mean_call = pl.pallas_call(
    _mean_body,
    out_shape=jax.ShapeDtypeStruct((N, D), jnp.float32),
    grid=(pl.cdiv(N, MB),),
    in_specs=[pl.BlockSpec((SLABS, MB, SW), lambda i: (0, i, 0))] * 3,
    out_specs=pl.BlockSpec((MB, D), lambda i: (i, 0)),
)


def kernel(centroid_embs, finetune_embs, assign_val, ft_assign_val,
           edge_weight, assign_idx, ft_assign_idx, edge_index):
    ft_idx = ft_assign_idx.astype(jnp.int32).reshape(-1)
    ft_val = ft_assign_val.reshape(-1)
    refined = _refine_kernel(centroid_embs, finetune_embs, ft_val, ft_idx)

    idx = jnp.pad(assign_idx.astype(jnp.int32), ((0, N_PAD - N), (0, 0)))
    val = jnp.pad(assign_val, ((0, N_PAD - N), (0, 0)))
    x0s = _compose_kernel(refined, val.reshape(-1),
                          idx.reshape(-1, X0_EC * K1))

    src = jnp.pad(edge_index[0].astype(jnp.int32),
                  (0, E_PAD - E)).reshape(-1, PR_EC)
    dst = jnp.pad(edge_index[1].astype(jnp.int32),
                  (0, E_PAD - E)).reshape(-1, PR_EC)
    w = jnp.pad(edge_weight, (0, E_PAD - E))
    x1s = _prop_kernel(x0s.reshape(-1, SW), src, dst, w)
    x2s = _prop_kernel(x1s.reshape(-1, SW), src, dst, w)

    return _mean_call(x0s, x1s, x2s)
